# Initial kernel scaffold; baseline (speedup 1.0000x reference)
import jax
import jax.numpy as jnp
from jax.experimental import pallas as pl


def kernel(x, Wr, W1, b1, W2, b2):
    raise NotImplementedError("write your pallas kernel here")


# speedup vs baseline: 1.3286x; 1.3286x over previous
import jax
import jax.numpy as jnp
from jax import lax
from jax.experimental import pallas as pl

E = 8
D = 1024
F = 2048
T = 2048
BT = 1024
NEG = -1e30


def _router_body(x_ref, wrp_ref, comb_ref):
    lg = jnp.dot(x_ref[...], wrp_ref[...], preferred_element_type=jnp.float32)
    col = lax.broadcasted_iota(jnp.int32, lg.shape, 1)
    lg = jnp.where(col < E, lg, NEG)
    m1 = jnp.max(lg, axis=1, keepdims=True)
    i1 = jnp.min(jnp.where(lg == m1, col, 128), axis=1, keepdims=True)
    lg2 = jnp.where(col == i1, NEG, lg)
    m2 = jnp.max(lg2, axis=1, keepdims=True)
    i2 = jnp.min(jnp.where(lg2 == m2, col, 128), axis=1, keepdims=True)
    g1 = 1.0 / (1.0 + jnp.exp(m2 - m1))
    g2 = 1.0 - g1
    comb_ref[...] = jnp.where(col == i1, g1, 0.0) + jnp.where(col == i2, g2, 0.0)


def _dense_body(comb_ref, x_ref, w1_ref, b1_ref, w2_ref, b2_ref, out_ref):
    e = pl.program_id(1)
    col = lax.broadcasted_iota(jnp.int32, (BT, 128), 1)
    c = jnp.sum(jnp.where(col == e, comb_ref[...], 0.0), axis=1, keepdims=True)
    xb = x_ref[...].astype(jnp.bfloat16)
    h1 = jnp.dot(xb, w1_ref[0], preferred_element_type=jnp.float32) + b1_ref[0]
    g = jax.nn.gelu(h1)
    h2 = jnp.dot(g.astype(jnp.bfloat16), w2_ref[0],
                 preferred_element_type=jnp.float32) + b2_ref[0]
    contrib = c * h2

    @pl.when(e == 0)
    def _():
        out_ref[...] = contrib

    @pl.when(e > 0)
    def _():
        out_ref[...] += contrib


def kernel(x, Wr, W1, b1, W2, b2):
    wr_pad = jnp.zeros((D, 128), jnp.float32).at[:, :E].set(Wr)
    comb = pl.pallas_call(
        _router_body,
        out_shape=jax.ShapeDtypeStruct((T, 128), jnp.float32),
    )(x, wr_pad)

    w1b = W1.astype(jnp.bfloat16)
    w2b = W2.astype(jnp.bfloat16)
    out = pl.pallas_call(
        _dense_body,
        grid=(T // BT, E),
        in_specs=[
            pl.BlockSpec((BT, 128), lambda i, e: (i, 0)),
            pl.BlockSpec((BT, D), lambda i, e: (i, 0)),
            pl.BlockSpec((1, D, F), lambda i, e: (e, 0, 0)),
            pl.BlockSpec((1, 1, F), lambda i, e: (e, 0, 0)),
            pl.BlockSpec((1, F, D), lambda i, e: (e, 0, 0)),
            pl.BlockSpec((1, 1, D), lambda i, e: (e, 0, 0)),
        ],
        out_specs=pl.BlockSpec((BT, D), lambda i, e: (i, 0)),
        out_shape=jax.ShapeDtypeStruct((T, D), jnp.float32),
    )(comb, x, w1b, b1.reshape(E, 1, F), w2b, b2.reshape(E, 1, D))
    return out


# speedup vs baseline: 1.4975x; 1.1271x over previous
import functools

import jax
import jax.numpy as jnp
from jax import lax
from jax.experimental import pallas as pl
from jax.experimental.pallas import tpu as pltpu
from jax.experimental.pallas import tpu_sc as plsc

E = 8
K = 2
D = 1024
F = 2048
T = 2048
A = K * T
BLK = 256
NB = A // BLK + E
NPAD = NB * BLK
CH = 512
NEG = -1e30

NW = 32


def _router_body(x_ref, wrp_ref, pos_ref, gates_ref, counts_ref):
    lg = jnp.dot(x_ref[...], wrp_ref[...], preferred_element_type=jnp.float32)
    col = lax.broadcasted_iota(jnp.int32, lg.shape, 1)
    lg = jnp.where(col < E, lg, NEG)
    m1 = jnp.max(lg, axis=1, keepdims=True)
    i1 = jnp.min(jnp.where(lg == m1, col, 128), axis=1, keepdims=True)
    lg2 = jnp.where(col == i1, NEG, lg)
    m2 = jnp.max(lg2, axis=1, keepdims=True)
    i2 = jnp.min(jnp.where(lg2 == m2, col, 128), axis=1, keepdims=True)
    g1 = 1.0 / (1.0 + jnp.exp(m2 - m1))
    g2 = 1.0 - g1

    r = lax.broadcasted_iota(jnp.int32, (CH, CH), 0)
    c = lax.broadcasted_iota(jnp.int32, (CH, CH), 1)
    tri = (c < r).astype(jnp.bfloat16)
    ccol = lax.broadcasted_iota(jnp.int32, (CH, 128), 1)
    carry = jnp.zeros((1, 128), jnp.float32)
    ranks, onehots = [], []
    n_per_k = T // CH
    for ch in range(A // CH):
        k = ch // n_per_k
        t0 = (ch % n_per_k) * CH
        e_chunk = (i1 if k == 0 else i2)[t0:t0 + CH]
        oh = (ccol == e_chunk).astype(jnp.float32)
        p = jnp.dot(tri, oh.astype(jnp.bfloat16),
                    preferred_element_type=jnp.float32) + carry
        ranks.append(jnp.sum(p * oh, axis=1, keepdims=True))
        onehots.append(oh)
        carry = carry + jnp.sum(oh, axis=0, keepdims=True)

    counts_ref[...] = carry
    pcb = jnp.ceil(carry / BLK)
    row = lax.broadcasted_iota(jnp.int32, (1, 128), 1)
    ps = jnp.zeros((1, 128), jnp.float32)
    for e in range(E):
        ps_e = jnp.sum(jnp.where(row < e, pcb, 0.0), axis=1, keepdims=True)
        ps = ps + jnp.where(row == e, ps_e, 0.0)
    ps = ps * BLK

    for ch in range(A // CH):
        ps_sel = jnp.sum(onehots[ch] * ps, axis=1, keepdims=True)
        pos_chunk = (ps_sel + ranks[ch]).astype(jnp.int32)
        pos_ref[ch * CH:(ch + 1) * CH, :] = pos_chunk
        k = ch // n_per_k
        t0 = (ch % n_per_k) * CH
        g_chunk = (g1 if k == 0 else g2)[t0:t0 + CH]
        gates_ref[ch * CH:(ch + 1) * CH, :] = g_chunk


def _sc_dispatch_body(x_hbm, pos_hbm, xs_hbm, posv, xv, sem):
    wid = lax.axis_index("s") * 2 + lax.axis_index("c")
    for j in range(2):
        a_base = wid * (A // NW) + j * 64
        t_base = lax.rem(a_base, T)
        pltpu.sync_copy(pos_hbm.at[pl.ds(a_base, 64)], posv)
        pltpu.sync_copy(x_hbm.at[pl.ds(t_base, 64)], xv)
        pltpu.async_copy(xv, xs_hbm.at[posv], sem).wait()


@functools.cache
def _sc_dispatch():
    return pl.kernel(
        _sc_dispatch_body,
        mesh=plsc.VectorSubcoreMesh(core_axis_name="c", subcore_axis_name="s"),
        out_type=jax.ShapeDtypeStruct((NPAD, D), jnp.float32),
        scratch_types=[
            pltpu.VMEM((64,), jnp.int32),
            pltpu.VMEM((64, D), jnp.float32),
            pltpu.SemaphoreType.DMA,
        ],
    )


def _ffn_body(s_ref, xs_ref, w1_ref, b1_ref, w2_ref, b2_ref, out_ref):
    b = pl.program_id(0)

    @pl.when(s_ref[b] < E)
    def _():
        xb = xs_ref[...].astype(jnp.bfloat16)
        h1 = jnp.dot(xb, w1_ref[0], preferred_element_type=jnp.float32)
        h1 = h1 + b1_ref[0]
        g = jax.nn.gelu(h1)
        h2 = jnp.dot(g.astype(jnp.bfloat16), w2_ref[0],
                     preferred_element_type=jnp.float32) + b2_ref[0]
        out_ref[...] = h2


def _sc_gather2_body(h_hbm, pos_hbm, h0_hbm, h1_hbm, p0v, p1v, r0v, r1v, sem):
    wid = lax.axis_index("s") * 2 + lax.axis_index("c")
    for j in range(2):
        t_base = wid * (T // NW) + j * 32
        pltpu.sync_copy(pos_hbm.at[pl.ds(t_base, 32)], p0v)
        pltpu.sync_copy(pos_hbm.at[pl.ds(T + t_base, 32)], p1v)
        pltpu.async_copy(h_hbm.at[p0v], r0v, sem).wait()
        pltpu.sync_copy(r0v, h0_hbm.at[pl.ds(t_base, 32)])
        pltpu.async_copy(h_hbm.at[p1v], r1v, sem).wait()
        pltpu.sync_copy(r1v, h1_hbm.at[pl.ds(t_base, 32)])


@functools.cache
def _sc_gather2():
    return pl.kernel(
        _sc_gather2_body,
        mesh=plsc.VectorSubcoreMesh(core_axis_name="c", subcore_axis_name="s"),
        out_type=(
            jax.ShapeDtypeStruct((T, D), jnp.float32),
            jax.ShapeDtypeStruct((T, D), jnp.float32),
        ),
        scratch_types=[
            pltpu.VMEM((32,), jnp.int32),
            pltpu.VMEM((32,), jnp.int32),
            pltpu.VMEM((32, D), jnp.float32),
            pltpu.VMEM((32, D), jnp.float32),
            pltpu.SemaphoreType.DMA,
        ],
    )


def _mix_body(h0_ref, h1_ref, g0_ref, g1_ref, out_ref):
    out_ref[...] = g0_ref[...] * h0_ref[...] + g1_ref[...] * h1_ref[...]


def kernel(x, Wr, W1, b1, W2, b2):
    wr_pad = jnp.zeros((D, 128), jnp.float32).at[:, :E].set(Wr)
    pos2d, gates2d, counts = pl.pallas_call(
        _router_body,
        out_shape=(
            jax.ShapeDtypeStruct((A, 1), jnp.int32),
            jax.ShapeDtypeStruct((A, 1), jnp.float32),
            jax.ShapeDtypeStruct((1, 128), jnp.float32),
        ),
    )(x, wr_pad)
    pos = pos2d.reshape(A)

    pc = jnp.ceil(counts[0, :E] / BLK).astype(jnp.int32) * BLK
    ps_incl = jnp.cumsum(pc)
    bexp = jnp.searchsorted(
        ps_incl, jnp.arange(NB, dtype=jnp.int32) * BLK, side="right"
    ).astype(jnp.int32)

    x_sorted = _sc_dispatch()(x, pos)

    w1b = W1.astype(jnp.bfloat16)
    w2b = W2.astype(jnp.bfloat16)
    h_sorted = pl.pallas_call(
        _ffn_body,
        grid_spec=pltpu.PrefetchScalarGridSpec(
            num_scalar_prefetch=1,
            grid=(NB,),
            in_specs=[
                pl.BlockSpec((BLK, D), lambda b, s: (b, 0)),
                pl.BlockSpec((1, D, F), lambda b, s: (jnp.minimum(s[b], E - 1), 0, 0)),
                pl.BlockSpec((1, 1, F), lambda b, s: (jnp.minimum(s[b], E - 1), 0, 0)),
                pl.BlockSpec((1, F, D), lambda b, s: (jnp.minimum(s[b], E - 1), 0, 0)),
                pl.BlockSpec((1, 1, D), lambda b, s: (jnp.minimum(s[b], E - 1), 0, 0)),
            ],
            out_specs=pl.BlockSpec((BLK, D), lambda b, s: (b, 0)),
        ),
        out_shape=jax.ShapeDtypeStruct((NPAD, D), jnp.float32),
    )(bexp, x_sorted, w1b, b1.reshape(E, 1, F), w2b, b2.reshape(E, 1, D))

    h0, h1 = _sc_gather2()(h_sorted, pos)
    return pl.pallas_call(
        _mix_body,
        out_shape=jax.ShapeDtypeStruct((T, D), jnp.float32),
    )(h0, h1, gates2d[:T], gates2d[T:])
